# baseline (speedup 1.0000x reference)
import functools
import math

import jax
import jax.numpy as jnp
from jax import lax
from jax.experimental import pallas as pl
from jax.experimental.pallas import tpu as pltpu
from jax.experimental.pallas import tpu_sc as plsc

D_MODEL = 64
LANES = 16
NUM_CORES = 2
NUM_SUBCORES = 16
NUM_WORKERS = NUM_CORES * NUM_SUBCORES
CHUNK = 128
NSLOTS = 6
OSLOTS = 3
SCALE = math.sqrt(D_MODEL)


def _sc_embed(xT, table, batch, hist):
    n_blocks = batch // CHUNK
    mesh = plsc.VectorSubcoreMesh(core_axis_name="c", subcore_axis_name="s")

    @functools.partial(
        pl.kernel,
        mesh=mesh,
        out_type=jax.ShapeDtypeStruct(
            (hist, 8, n_blocks, 8, 128), jnp.float32),
        scratch_types=[
            pltpu.VMEM((hist, CHUNK), jnp.int32),
            pltpu.VMEM((NSLOTS, CHUNK, D_MODEL), jnp.float32),
            pltpu.VMEM((OSLOTS, 8, 8, 129), jnp.float32),
            pltpu.SemaphoreType.DMA,
            pltpu.SemaphoreType.DMA,
        ],
        compiler_params=pltpu.CompilerParams(
            use_tc_tiling_on_sc=False, needs_layout_passes=False),
    )
    def k(xT_hbm, table_hbm, out_hbm, idx_v, rows_v, t5_v, gsem, osem):
        wid = lax.axis_index("s") * NUM_CORES + lax.axis_index("c")
        pltpu.sync_copy(xT_hbm.at[:, pl.ds(wid * CHUNK, CHUNK)], idx_v)

        def gather(h, slot):
            return pltpu.async_copy(
                table_hbm.at[idx_v.at[h]], rows_v.at[slot], gsem)

        for _p in range(4):
            gather(_p, _p)

        def transpose_scale(gslot, oslot):
            iota = lax.iota(jnp.int32, LANES)
            ti_half = jax.lax.shift_right_logical(iota, 3)
            r_vec = jax.lax.bitwise_and(iota, 7)
            dst = t5_v.at[oslot]

            @plsc.parallel_loop(0, CHUNK, unroll=8)
            def _c_loop(c):
                c16 = jnp.full((LANES,), 0, jnp.int32) + c
                for k in range(D_MODEL // LANES):
                    v = rows_v[gslot, c, pl.ds(k * LANES, LANES)]
                    plsc.store_scatter(
                        dst, [ti_half + (2 * k), r_vec, c16], v * SCALE)

        def body(h, carry):
            gslot = lax.rem(h, NSLOTS)
            oslot = lax.rem(h, OSLOTS)
            pltpu.make_async_copy(
                table_hbm.at[idx_v.at[h]], rows_v.at[gslot], gsem).wait()

            @pl.when(h >= OSLOTS)
            def _wait_out():
                pltpu.make_async_copy(
                    t5_v.at[oslot, :, :, pl.ds(0, 128)],
                    out_hbm.at[h, :, wid], osem).wait()

            transpose_scale(gslot, oslot)
            pltpu.async_copy(
                t5_v.at[oslot, :, :, pl.ds(0, 128)],
                out_hbm.at[h, :, wid], osem)

            @pl.when(h + 4 < hist)
            def _next_gather():
                gather(h + 4, lax.rem(h + 4, NSLOTS))
            return carry

        lax.fori_loop(0, hist, body, 0)

        for _ in range(OSLOTS):
            pltpu.make_async_copy(
                t5_v.at[0, :, :, pl.ds(0, 128)],
                out_hbm.at[0, :, wid], osem).wait()

    return k(xT, table)


def kernel(x, emb_weight):
    batch, hist = x.shape
    xT = x.astype(jnp.int32).T
    out5 = _sc_embed(xT, emb_weight, batch, hist)
    return out5.transpose(2, 4, 0, 1, 3).reshape(batch, hist, D_MODEL)
